# Initial kernel scaffold; baseline (speedup 1.0000x reference)
import jax
import jax.numpy as jnp
from jax.experimental import pallas as pl


def kernel(image, defocus):
    raise NotImplementedError("write your pallas kernel here")


# speedup vs baseline: 1.5926x; 1.5926x over previous
import numpy as np
import jax
import jax.numpy as jnp
from jax.experimental import pallas as pl
from jax.experimental.pallas import tpu as pltpu

_R = 5
_NEG = -1e9
_TH = 128
_HALO = _R + 11


def _live_taps():
    taps = []
    for dy in range(-_R, _R + 1):
        for dx in range(-_R, _R + 1):
            d2 = dy * dy + dx * dx
            if d2 < (_R + 0.5) ** 2:
                taps.append((dy, dx, float(np.sqrt(d2))))
    return taps


_TAPS = _live_taps()


def _bokeh_body(img_ref, d_ref, bokeh_ref, dd_ref, img_scr, r05_ref, bw_ref,
                di_ref, wc_ref):
    TH = bokeh_ref.shape[2]
    W = bokeh_ref.shape[3]
    C = bokeh_ref.shape[1]
    LR = TH + 2 * _R + 6
    t = pl.program_id(1)
    row0 = t * TH

    img_scr[...] = img_ref[0, :, pl.ds(row0, LR), :]
    d = d_ref[0, 0, pl.ds(row0, LR), :]
    r = jnp.abs(d)
    r05_ref[...] = r + 0.5
    bw_ref[...] = 1.0 / (jnp.pi * r * r + 1.0)
    di_ref[...] = d.astype(jnp.int32).astype(jnp.float32)

    w0 = bw_ref[_R:_R + TH, _R:_R + W]
    for c in range(C):
        bokeh_ref[0, c] = w0 * img_scr[c, _R:_R + TH, _R:_R + W]
    wc_ref[...] = w0
    dd_ref[0, 0] = di_ref[_R:_R + TH, _R:_R + W]

    for dy, dx, dist in _TAPS:
        if dy == 0 and dx == 0:
            continue
        sy = _R - dy
        sx = _R - dx
        m = r05_ref[sy:sy + TH, sx:sx + W] >= dist
        w = jnp.where(m, bw_ref[sy:sy + TH, sx:sx + W], 0.0)
        for c in range(C):
            bokeh_ref[0, c] = bokeh_ref[0, c] + w * img_scr[c, sy:sy + TH,
                                                            sx:sx + W]
        wc_ref[...] = wc_ref[...] + w
        dd_ref[0, 0] = jnp.maximum(
            dd_ref[0, 0], jnp.where(m, di_ref[sy:sy + TH, sx:sx + W], _NEG))

    inv = 1.0 / wc_ref[...]
    for c in range(C):
        bokeh_ref[0, c] = bokeh_ref[0, c] * inv


@jax.jit
def kernel(image, defocus):
    B, C, H, W = image.shape
    Hp = H + _R + _HALO
    Wp = W + 2 * _R
    LR = _TH + 2 * _R + 6
    img_p = jnp.pad(image, ((0, 0), (0, 0), (_R, _HALO), (_R, _R)))
    d_p = jnp.pad(defocus, ((0, 0), (0, 0), (_R, _HALO), (_R, _R)))
    T = H // _TH
    bokeh, dd = pl.pallas_call(
        _bokeh_body,
        grid=(B, T),
        in_specs=[
            pl.BlockSpec((1, C, Hp, Wp), lambda b, t: (b, 0, 0, 0)),
            pl.BlockSpec((1, 1, Hp, Wp), lambda b, t: (b, 0, 0, 0)),
        ],
        out_specs=[
            pl.BlockSpec((1, C, _TH, W), lambda b, t: (b, 0, t, 0)),
            pl.BlockSpec((1, 1, _TH, W), lambda b, t: (b, 0, t, 0)),
        ],
        out_shape=[
            jax.ShapeDtypeStruct((B, C, H, W), jnp.float32),
            jax.ShapeDtypeStruct((B, 1, H, W), jnp.float32),
        ],
        scratch_shapes=[
            pltpu.VMEM((C, LR, Wp), jnp.float32),
            pltpu.VMEM((LR, Wp), jnp.float32),
            pltpu.VMEM((LR, Wp), jnp.float32),
            pltpu.VMEM((LR, Wp), jnp.float32),
            pltpu.VMEM((_TH, W), jnp.float32),
        ],
    )(img_p, d_p)
    return bokeh, dd


# speedup vs baseline: 1.5991x; 1.0041x over previous
import numpy as np
import jax
import jax.numpy as jnp
from jax.experimental import pallas as pl
from jax.experimental.pallas import tpu as pltpu

_R = 5
_NEG = -1e9
_TH = 128
_HALO = _R + 11


def _tap_classes():
    by_d2 = {}
    for dy in range(-_R, _R + 1):
        for dx in range(-_R, _R + 1):
            d2 = dy * dy + dx * dx
            if 0 < d2 < (_R + 0.5) ** 2:
                by_d2.setdefault(d2, []).append((dy, dx))
    return [(float(np.sqrt(d2)), by_d2[d2]) for d2 in sorted(by_d2)]


_CLASSES = _tap_classes()


def _bokeh_body(img_ref, d_ref, bokeh_ref, dd_ref, img_scr, wpl_ref, cand_ref,
                wc_ref):
    TH = bokeh_ref.shape[2]
    W = bokeh_ref.shape[3]
    LR = TH + 2 * _R + 6
    t = pl.program_id(1)
    row0 = t * TH

    img_scr[...] = img_ref[0, :, pl.ds(row0, LR), :]
    d = d_ref[0, 0, pl.ds(row0, LR), :]
    r = jnp.abs(d)
    r05 = r + 0.5
    bw = 1.0 / (jnp.pi * r * r + 1.0)
    di = d.astype(jnp.int32).astype(jnp.float32)
    for k, (dist, _) in enumerate(_CLASSES):
        m = r05 >= dist
        wpl_ref[k] = jnp.where(m, bw, 0.0)
        cand_ref[k] = jnp.where(m, di, _NEG)

    w0 = bw[_R:_R + TH, _R:_R + W]
    bokeh_ref[0] = w0[None] * img_scr[:, _R:_R + TH, _R:_R + W]
    wc_ref[...] = w0
    dd_ref[0, 0] = di[_R:_R + TH, _R:_R + W]

    for k, (dist, taps) in enumerate(_CLASSES):
        gacc = None
        for dy, dx in taps:
            sy = _R - dy
            sx = _R - dx
            w = wpl_ref[k, sy:sy + TH, sx:sx + W]
            contrib = w[None] * img_scr[:, sy:sy + TH, sx:sx + W]
            cnd = cand_ref[k, sy:sy + TH, sx:sx + W]
            if gacc is None:
                gacc, gwc, gdd = contrib, w, cnd
            else:
                gacc = gacc + contrib
                gwc = gwc + w
                gdd = jnp.maximum(gdd, cnd)
        bokeh_ref[0] = bokeh_ref[0] + gacc
        wc_ref[...] = wc_ref[...] + gwc
        dd_ref[0, 0] = jnp.maximum(dd_ref[0, 0], gdd)

    inv = 1.0 / wc_ref[...]
    bokeh_ref[0] = bokeh_ref[0] * inv[None]


@jax.jit
def kernel(image, defocus):
    B, C, H, W = image.shape
    Hp = H + _R + _HALO
    Wp = W + 2 * _R
    LR = _TH + 2 * _R + 6
    K = len(_CLASSES)
    img_p = jnp.pad(image, ((0, 0), (0, 0), (_R, _HALO), (_R, _R)))
    d_p = jnp.pad(defocus, ((0, 0), (0, 0), (_R, _HALO), (_R, _R)))
    T = H // _TH
    bokeh, dd = pl.pallas_call(
        _bokeh_body,
        grid=(B, T),
        in_specs=[
            pl.BlockSpec((1, C, Hp, Wp), lambda b, t: (b, 0, 0, 0)),
            pl.BlockSpec((1, 1, Hp, Wp), lambda b, t: (b, 0, 0, 0)),
        ],
        out_specs=[
            pl.BlockSpec((1, C, _TH, W), lambda b, t: (b, 0, t, 0)),
            pl.BlockSpec((1, 1, _TH, W), lambda b, t: (b, 0, t, 0)),
        ],
        out_shape=[
            jax.ShapeDtypeStruct((B, C, H, W), jnp.float32),
            jax.ShapeDtypeStruct((B, 1, H, W), jnp.float32),
        ],
        scratch_shapes=[
            pltpu.VMEM((C, LR, Wp), jnp.float32),
            pltpu.VMEM((K, LR, Wp), jnp.float32),
            pltpu.VMEM((K, LR, Wp), jnp.float32),
            pltpu.VMEM((_TH, W), jnp.float32),
        ],
    )(img_p, d_p)
    return bokeh, dd


# speedup vs baseline: 5.0784x; 3.1758x over previous
import numpy as np
import jax
import jax.numpy as jnp
from jax.experimental import pallas as pl
from jax.experimental.pallas import tpu as pltpu

_R = 5
_NEG = -1e9
_TH = 128
_HALO = _R + 11

_D2S = sorted({dy * dy + dx * dx
               for dy in range(-_R, _R + 1) for dx in range(-_R, _R + 1)
               if dy * dy + dx * dx < (_R + 0.5) ** 2})
_KOF = {d2: k for k, d2 in enumerate(_D2S)}
_YMAX = [int(np.floor(np.sqrt((_R + 0.5) ** 2 - 1e-9 - dx * dx)))
         for dx in range(_R + 1)]


def _bokeh_body(img_ref, d_ref, bokeh_ref, dd_ref, img_scr, wpl_ref, cand_ref,
                b_ref, wc_ref):
    TH = bokeh_ref.shape[2]
    W = bokeh_ref.shape[3]
    LR = TH + 2 * _R + 6
    t = pl.program_id(1)
    row0 = t * TH

    img_scr[...] = img_ref[0, :, pl.ds(row0, LR), :]
    d = d_ref[0, 0, pl.ds(row0, LR), :]
    r = jnp.abs(d)
    r05 = r + 0.5
    bw = 1.0 / (jnp.pi * r * r + 1.0)
    di = d.astype(jnp.int32).astype(jnp.float32)
    wpl_ref[0] = bw
    cand_ref[0] = di
    for d2 in _D2S[1:]:
        m = r05 >= float(np.sqrt(d2))
        wpl_ref[_KOF[d2]] = jnp.where(m, bw, 0.0)
        cand_ref[_KOF[d2]] = jnp.where(m, di, _NEG)

    first = True
    for adx in range(_R + 1):
        bacc = bwc = bdd = None
        for dy in range(-_YMAX[adx], _YMAX[adx] + 1):
            k = _KOF[dy * dy + adx * adx]
            sy = _R - dy
            w = wpl_ref[k, sy:sy + TH, :]
            cnd = cand_ref[k, sy:sy + TH, :]
            im = img_scr[:, sy:sy + TH, :]
            if bacc is None:
                bacc = w[None] * im
                bwc = w
                bdd = cnd
            else:
                bacc = bacc + w[None] * im
                bwc = bwc + w
                bdd = jnp.maximum(bdd, cnd)
        b_ref[0:3] = bacc
        b_ref[3] = bwc
        b_ref[4] = bdd
        for dx in sorted({adx, -adx}):
            sx = _R - dx
            if first:
                bokeh_ref[0] = b_ref[0:3, :, sx:sx + W]
                wc_ref[...] = b_ref[3, :, sx:sx + W]
                dd_ref[0, 0] = b_ref[4, :, sx:sx + W]
                first = False
            else:
                bokeh_ref[0] = bokeh_ref[0] + b_ref[0:3, :, sx:sx + W]
                wc_ref[...] = wc_ref[...] + b_ref[3, :, sx:sx + W]
                dd_ref[0, 0] = jnp.maximum(dd_ref[0, 0],
                                           b_ref[4, :, sx:sx + W])

    inv = 1.0 / wc_ref[...]
    bokeh_ref[0] = bokeh_ref[0] * inv[None]


@jax.jit
def kernel(image, defocus):
    B, C, H, W = image.shape
    Hp = H + _R + _HALO
    Wp = W + 2 * _R
    LR = _TH + 2 * _R + 6
    K = len(_D2S)
    img_p = jnp.pad(image, ((0, 0), (0, 0), (_R, _HALO), (_R, _R)))
    d_p = jnp.pad(defocus, ((0, 0), (0, 0), (_R, _HALO), (_R, _R)))
    T = H // _TH
    bokeh, dd = pl.pallas_call(
        _bokeh_body,
        grid=(B, T),
        in_specs=[
            pl.BlockSpec((1, C, Hp, Wp), lambda b, t: (b, 0, 0, 0)),
            pl.BlockSpec((1, 1, Hp, Wp), lambda b, t: (b, 0, 0, 0)),
        ],
        out_specs=[
            pl.BlockSpec((1, C, _TH, W), lambda b, t: (b, 0, t, 0)),
            pl.BlockSpec((1, 1, _TH, W), lambda b, t: (b, 0, t, 0)),
        ],
        out_shape=[
            jax.ShapeDtypeStruct((B, C, H, W), jnp.float32),
            jax.ShapeDtypeStruct((B, 1, H, W), jnp.float32),
        ],
        scratch_shapes=[
            pltpu.VMEM((C, LR, Wp), jnp.float32),
            pltpu.VMEM((K, LR, Wp), jnp.float32),
            pltpu.VMEM((K, LR, Wp), jnp.float32),
            pltpu.VMEM((5, _TH, Wp), jnp.float32),
            pltpu.VMEM((_TH, W), jnp.float32),
        ],
    )(img_p, d_p)
    return bokeh, dd
